# baseline (speedup 1.0000x reference)
import dataclasses

import jax
import jax.numpy as jnp
from jax import lax
from jax.experimental import pallas as pl
from jax.experimental.pallas import tpu as pltpu
from jax.experimental.pallas import tpu_sc as plsc

NUM_CLASSES_ = 1000
PAD_CLASSES = 1024
N_ROWS = 16384
N_TILES = 32
ROWS_PER_TILE = N_ROWS // N_TILES
CHUNK_ROWS = 16
CHUNKS_PER_TILE = ROWS_PER_TILE // CHUNK_ROWS
NBUF = 2


TC_TILES = 1
SC_TILES = N_TILES - TC_TILES
TC_ROWS = TC_TILES * ROWS_PER_TILE


def _one_hot_body(idx_hbm, onoff_hbm, out_hbm, idx_v, onoff_v, buf, sem0, sem1):
    c = lax.axis_index("c")
    s = lax.axis_index("s")
    wid = c * 16 + s

    @pl.when(wid < SC_TILES)
    def _():
        _one_hot_tile(wid, idx_hbm, onoff_hbm, out_hbm, idx_v, onoff_v, buf,
                      sem0, sem1)


def _one_hot_tile(wid, idx_hbm, onoff_hbm, out_hbm, idx_v, onoff_v, buf, sem0, sem1):
    row_base = wid * ROWS_PER_TILE

    pltpu.sync_copy(idx_hbm.at[pl.ds(row_base, ROWS_PER_TILE)], idx_v)
    pltpu.sync_copy(onoff_hbm, onoff_v)
    on_vec = onoff_v[pl.ds(0, 16)]
    off_vec = onoff_v[pl.ds(16, 16)]
    lane_iota = lax.iota(jnp.int32, 16)

    @pl.loop(0, NBUF * CHUNK_ROWS, step=1)
    def _(r):
        @pl.loop(0, 960, step=128)
        def _(j):
            for d in range(8):
                buf[r, pl.ds(j + 16 * d, 16)] = off_vec
        for d in range(2):
            buf[r, pl.ds(960 + 16 * d, 16)] = off_vec
        buf[r, pl.ds(NUM_CLASSES_ - 16, 16)] = off_vec

    sems = (sem0, sem1)

    def chunk_cols(i):
        return idx_v[pl.ds(i * CHUNK_ROWS, 16)]

    def chunk_rows(slot):
        return lane_iota + slot * CHUNK_ROWS

    def copy_desc(i, slot):
        return pltpu.make_async_copy(
            buf.at[pl.ds(slot * CHUNK_ROWS, CHUNK_ROWS), :],
            out_hbm.at[pl.ds(row_base + i * CHUNK_ROWS, CHUNK_ROWS), :],
            sems[slot],
        )

    def issue(i, slot):
        plsc.store_scatter(buf, [chunk_rows(slot), chunk_cols(i)], on_vec)
        copy_desc(i, slot).start()

    def drain_restore(i, slot):
        copy_desc(i, slot).wait()
        plsc.store_scatter(buf, [chunk_rows(slot), chunk_cols(i)], off_vec)

    for d in range(NBUF):
        issue(d, d)

    @pl.loop(NBUF, CHUNKS_PER_TILE, step=NBUF)
    def _(i):
        for d in range(NBUF):
            drain_restore(i + d - NBUF, d)
            issue(i + d, d)

    for d in range(NBUF):
        copy_desc(CHUNKS_PER_TILE - NBUF + d, d).wait()


def _tc_tail_body(big_ref, idx_ref, on_ref, off_ref, out_ref):
    del big_ref
    ids = idx_ref[...]
    cols = lax.broadcasted_iota(jnp.int32, (TC_ROWS, NUM_CLASSES_), 1)
    out_ref[...] = jnp.where(cols == ids, on_ref[0], off_ref[0])


def kernel(inputs, on_value, off_value):
    onoff = jnp.concatenate([
        jnp.broadcast_to(on_value.astype(jnp.float32), (16,)),
        jnp.broadcast_to(off_value.astype(jnp.float32), (16,)),
    ])
    mesh = plsc.VectorSubcoreMesh(
        core_axis_name="c", subcore_axis_name="s", num_cores=2, num_subcores=16
    )
    cp = pltpu.CompilerParams(use_tc_tiling_on_sc=True, skip_device_barrier=True)
    if "needs_layout_passes" in pltpu.CompilerParams.__dataclass_fields__:
        cp = dataclasses.replace(cp, needs_layout_passes=False)
    k = pl.kernel(
        _one_hot_body,
        out_type=jax.ShapeDtypeStruct((N_ROWS, NUM_CLASSES_), jnp.float32),
        mesh=mesh,
        compiler_params=cp,
        scratch_types=[
            pltpu.VMEM((ROWS_PER_TILE,), jnp.int32),
            pltpu.VMEM((2 * 16,), jnp.float32),
            pltpu.VMEM((NBUF * CHUNK_ROWS, NUM_CLASSES_), jnp.float32),
            pltpu.SemaphoreType.DMA,
            pltpu.SemaphoreType.DMA,
        ],
    )
    idx32 = inputs.astype(jnp.int32)
    sc_out = k(idx32, onoff)

    tail = pl.pallas_call(
        _tc_tail_body,
        out_shape=jax.ShapeDtypeStruct((N_ROWS, NUM_CLASSES_), jnp.float32),
        grid=(1,),
        in_specs=[
            pl.BlockSpec(memory_space=pl.ANY),
            pl.BlockSpec((TC_ROWS, 1), lambda i: (N_TILES - TC_TILES, 0)),
            pl.BlockSpec(memory_space=pltpu.SMEM),
            pl.BlockSpec(memory_space=pltpu.SMEM),
        ],
        out_specs=pl.BlockSpec((TC_ROWS, NUM_CLASSES_),
                               lambda i: (N_TILES - TC_TILES, 0)),
        input_output_aliases={0: 0},
    )
    return tail(
        sc_out,
        idx32.reshape(N_ROWS, 1),
        on_value.astype(jnp.float32).reshape(1),
        off_value.astype(jnp.float32).reshape(1),
    )


# speedup vs baseline: 2.1559x; 2.1559x over previous
import dataclasses

import jax
import jax.numpy as jnp
from jax import lax
from jax.experimental import pallas as pl
from jax.experimental.pallas import tpu as pltpu
from jax.experimental.pallas import tpu_sc as plsc

NUM_CLASSES_ = 1000
N_ROWS = 16384
N_TILES = 32
COLS_PER_TILE = N_ROWS // N_TILES
CHUNK_CLASSES = 40
CHUNKS_PER_TILE = NUM_CLASSES_ // CHUNK_CLASSES
GROUPS = COLS_PER_TILE // 16
NBUF = 2


def _one_hot_body(idx_hbm, onoff_hbm, out_hbm, idx_v, onoff_v, buf, sem0, sem1):
    c = lax.axis_index("c")
    s = lax.axis_index("s")
    wid = c * 16 + s
    col_base = wid * COLS_PER_TILE

    pltpu.sync_copy(idx_hbm.at[pl.ds(col_base, COLS_PER_TILE)], idx_v)
    pltpu.sync_copy(onoff_hbm, onoff_v)
    on_vec = onoff_v[pl.ds(0, 16)]
    off_vec = onoff_v[pl.ds(16, 16)]
    lane_iota = lax.iota(jnp.int32, 16)

    @pl.loop(0, NBUF * CHUNK_CLASSES, step=1)
    def _(r):
        @pl.loop(0, COLS_PER_TILE, step=128)
        def _(j):
            for d in range(8):
                buf[r, pl.ds(j + 16 * d, 16)] = off_vec

    sems = (sem0, sem1)

    def scatter(i, slot, val_vec):
        c0 = i * CHUNK_CLASSES
        for g in range(GROUPS):
            ids = idx_v[pl.ds(g * 16, 16)]
            rel = ids - c0
            mask = (rel >= 0) & (rel < CHUNK_CLASSES)
            rows = jnp.minimum(jnp.maximum(rel, 0), CHUNK_CLASSES - 1)
            rows = rows + slot * CHUNK_CLASSES
            cols = lane_iota + g * 16
            plsc.store_scatter(buf, [rows, cols], val_vec, mask=mask)

    def copy_desc(i, slot):
        return pltpu.make_async_copy(
            buf.at[pl.ds(slot * CHUNK_CLASSES, CHUNK_CLASSES), :],
            out_hbm.at[pl.ds(i * CHUNK_CLASSES, CHUNK_CLASSES),
                       pl.ds(col_base, COLS_PER_TILE)],
            sems[slot],
        )

    def issue(i, slot):
        scatter(i, slot, on_vec)
        copy_desc(i, slot).start()

    def drain_restore(i, slot):
        copy_desc(i, slot).wait()
        scatter(i, slot, off_vec)

    for d in range(NBUF):
        issue(d, d)

    @pl.loop(NBUF, CHUNKS_PER_TILE - 1, step=NBUF)
    def _(i):
        for d in range(NBUF):
            drain_restore(i + d - NBUF, d)
            issue(i + d, d)

    drain_restore(CHUNKS_PER_TILE - 3, 0)
    issue(CHUNKS_PER_TILE - 1, 0)
    copy_desc(CHUNKS_PER_TILE - 2, 1).wait()
    copy_desc(CHUNKS_PER_TILE - 1, 0).wait()


def kernel(inputs, on_value, off_value):
    onoff = jnp.concatenate([
        jnp.broadcast_to(on_value.astype(jnp.float32), (16,)),
        jnp.broadcast_to(off_value.astype(jnp.float32), (16,)),
    ])
    mesh = plsc.VectorSubcoreMesh(
        core_axis_name="c", subcore_axis_name="s", num_cores=2, num_subcores=16
    )
    cp = pltpu.CompilerParams(use_tc_tiling_on_sc=True, skip_device_barrier=True)
    if "needs_layout_passes" in pltpu.CompilerParams.__dataclass_fields__:
        cp = dataclasses.replace(cp, needs_layout_passes=False)
    k = pl.kernel(
        _one_hot_body,
        out_type=jax.ShapeDtypeStruct((NUM_CLASSES_, N_ROWS), jnp.float32),
        mesh=mesh,
        compiler_params=cp,
        scratch_types=[
            pltpu.VMEM((COLS_PER_TILE,), jnp.int32),
            pltpu.VMEM((2 * 16,), jnp.float32),
            pltpu.VMEM((NBUF * CHUNK_CLASSES, COLS_PER_TILE), jnp.float32),
            pltpu.SemaphoreType.DMA,
            pltpu.SemaphoreType.DMA,
        ],
    )
    out_t = k(inputs.astype(jnp.int32), onoff)
    return out_t.T


# speedup vs baseline: 2.2137x; 1.0268x over previous
import dataclasses

import jax
import jax.numpy as jnp
from jax import lax
from jax.experimental import pallas as pl
from jax.experimental.pallas import tpu as pltpu
from jax.experimental.pallas import tpu_sc as plsc

NUM_CLASSES_ = 1000
N_ROWS = 16384
N_TILES = 32
COLS_PER_TILE = N_ROWS // N_TILES
CHUNK_CLASSES = 200
CHUNK_COLS = 256
COL_HALVES = COLS_PER_TILE // CHUNK_COLS
CLASS_CHUNKS = NUM_CLASSES_ // CHUNK_CLASSES
CHUNKS_PER_TILE = CLASS_CHUNKS * COL_HALVES
GROUPS = CHUNK_COLS // 16
NBUF = 2


def _one_hot_body(idx_hbm, onoff_hbm, out_hbm, idx_v, onoff_v, buf, sem0, sem1):
    c = lax.axis_index("c")
    s = lax.axis_index("s")
    wid = c * 16 + s
    col_base = wid * COLS_PER_TILE

    pltpu.sync_copy(idx_hbm.at[pl.ds(col_base, COLS_PER_TILE)], idx_v)
    pltpu.sync_copy(onoff_hbm, onoff_v)
    on_vec = onoff_v[pl.ds(0, 16)]
    off_vec = onoff_v[pl.ds(16, 16)]
    lane_iota = lax.iota(jnp.int32, 16)

    @pl.loop(0, NBUF * CHUNK_CLASSES, step=1)
    def _(r):
        @pl.loop(0, CHUNK_COLS, step=128)
        def _(j):
            for d in range(8):
                buf[r, pl.ds(j + 16 * d, 16)] = off_vec

    sems = (sem0, sem1)

    def scatter(i, slot, val_vec):
        ci = i // COL_HALVES
        h = i % COL_HALVES
        c0 = ci * CHUNK_CLASSES
        for g in range(GROUPS):
            ids = idx_v[pl.ds(h * CHUNK_COLS + g * 16, 16)]
            rel = ids - c0
            mask = plsc.bitcast(rel, jnp.uint32) < CHUNK_CLASSES
            rows = rel + slot * CHUNK_CLASSES
            cols = lane_iota + g * 16
            plsc.store_scatter(buf, [rows, cols], val_vec, mask=mask)

    def copy_desc(i, slot):
        ci = i // COL_HALVES
        h = i % COL_HALVES
        return pltpu.make_async_copy(
            buf.at[pl.ds(slot * CHUNK_CLASSES, CHUNK_CLASSES), :],
            out_hbm.at[pl.ds(ci * CHUNK_CLASSES, CHUNK_CLASSES),
                       pl.ds(col_base + h * CHUNK_COLS, CHUNK_COLS)],
            sems[slot],
        )

    def issue(i, slot):
        scatter(i, slot, on_vec)
        copy_desc(i, slot).start()

    def drain_restore(i, slot):
        copy_desc(i, slot).wait()
        scatter(i, slot, off_vec)

    for d in range(NBUF):
        issue(d, d)

    for i in range(NBUF, CHUNKS_PER_TILE, NBUF):
        for d in range(NBUF):
            drain_restore(i + d - NBUF, d)
            issue(i + d, d)

    for d in range(NBUF):
        copy_desc(CHUNKS_PER_TILE - NBUF + d, d).wait()


def kernel(inputs, on_value, off_value):
    onoff = jnp.concatenate([
        jnp.broadcast_to(on_value.astype(jnp.float32), (16,)),
        jnp.broadcast_to(off_value.astype(jnp.float32), (16,)),
    ])
    mesh = plsc.VectorSubcoreMesh(
        core_axis_name="c", subcore_axis_name="s", num_cores=2, num_subcores=16
    )
    cp = pltpu.CompilerParams(use_tc_tiling_on_sc=True, skip_device_barrier=True)
    if "needs_layout_passes" in pltpu.CompilerParams.__dataclass_fields__:
        cp = dataclasses.replace(cp, needs_layout_passes=False)
    k = pl.kernel(
        _one_hot_body,
        out_type=jax.ShapeDtypeStruct((NUM_CLASSES_, N_ROWS), jnp.float32),
        mesh=mesh,
        compiler_params=cp,
        scratch_types=[
            pltpu.VMEM((COLS_PER_TILE,), jnp.int32),
            pltpu.VMEM((2 * 16,), jnp.float32),
            pltpu.VMEM((NBUF * CHUNK_CLASSES, CHUNK_COLS), jnp.float32),
            pltpu.SemaphoreType.DMA,
            pltpu.SemaphoreType.DMA,
        ],
    )
    out_t = k(inputs.astype(jnp.int32), onoff)
    return out_t.T
